# baseline (speedup 1.0000x reference)
import jax
import jax.numpy as jnp
from jax import lax
from jax.experimental import pallas as pl
from jax.experimental.pallas import tpu as pltpu

_SEM = type(pltpu.CORE_PARALLEL)


def _tile_body(x_ref, w_ref, b_ref, o_ref):
    o_ref[...] = (
        lax.dot_general(
            x_ref[...],
            w_ref[...],
            dimension_numbers=(((1,), (1,)), ((), ())),
            preferred_element_type=jnp.float32,
        )
        + b_ref[...]
    )


def _alloc_body(o_ref):
    pass


@jax.jit
def _linear(x, weight, bias):
    M, K = x.shape
    N, Kw = weight.shape
    assert K == Kw, "weight inner dim must match x"

    xb = x.astype(jnp.bfloat16)
    wb = weight.astype(jnp.bfloat16)
    b2d = bias.reshape(1, N).astype(jnp.float32)

    out0 = pl.pallas_call(
        _alloc_body,
        out_shape=jax.ShapeDtypeStruct((M, N), jnp.float32),
        out_specs=pl.BlockSpec(memory_space=pl.MemorySpace.ANY),
    )()

    bm = 512 if M % 512 == 0 else M
    bn = 2048 if N % 2048 == 0 else N
    grid = (N // bn, M // bm)

    mesh = pltpu.create_tensorcore_mesh("core", num_cores=2)

    def run(refs):
        x_ref, w_ref, b_ref, o_ref = refs

        @pl.core_map(mesh)
        def _():
            pltpu.emit_pipeline(
                _tile_body,
                grid=grid,
                in_specs=[
                    pl.BlockSpec((bm, K), lambda j, i: (i, 0)),
                    pl.BlockSpec((bn, K), lambda j, i: (j, 0)),
                    pl.BlockSpec((1, bn), lambda j, i: (0, j)),
                ],
                out_specs=[pl.BlockSpec((bm, bn), lambda j, i: (i, j))],
                core_axis_name="core",
                dimension_semantics=(_SEM.PARALLEL, _SEM.ARBITRARY),
            )(x_ref, w_ref, b_ref, o_ref)

    _, _, _, out = pl.run_state(run)((xb, wb, b2d, out0))
    return out


def kernel(x, weight, bias):
    return _linear(x, weight, bias)


# speedup vs baseline: 1.2146x; 1.2146x over previous
import jax
import jax.numpy as jnp
from jax import lax
from jax.experimental import pallas as pl
from jax.experimental.pallas import tpu as pltpu


def _tile_body(x_ref, w_ref, b_ref, o_ref):
    o_ref[...] = (
        lax.dot_general(
            x_ref[...].astype(jnp.bfloat16),
            w_ref[...],
            dimension_numbers=(((1,), (1,)), ((), ())),
            preferred_element_type=jnp.float32,
        )
        + b_ref[...]
    )


@jax.jit
def _linear(x, weight, bias):
    M, K = x.shape
    N, Kw = weight.shape
    assert K == Kw, "weight inner dim must match x"

    wb = weight.astype(jnp.bfloat16)
    b2d = bias.reshape(1, N).astype(jnp.float32)

    bm = 512 if M % 512 == 0 else M
    bn = 2048 if N % 2048 == 0 else N
    grid = (N // bn, M // bm)

    cost = pl.CostEstimate(
        flops=2 * M * N * K,
        transcendentals=0,
        bytes_accessed=4 * M * K * (N // bn) + 2 * N * K + 4 * (M * N + N),
    )

    return pl.pallas_call(
        _tile_body,
        out_shape=jax.ShapeDtypeStruct((M, N), jnp.float32),
        grid=grid,
        in_specs=[
            pl.BlockSpec((bm, K), lambda j, i: (i, 0)),
            pl.BlockSpec((bn, K), lambda j, i: (j, 0)),
            pl.BlockSpec((1, bn), lambda j, i: (0, j)),
        ],
        out_specs=pl.BlockSpec((bm, bn), lambda j, i: (i, j)),
        compiler_params=pltpu.CompilerParams(
            dimension_semantics=("parallel", "arbitrary"),
            vmem_limit_bytes=61 * 1024 * 1024,
        ),
        cost_estimate=cost,
    )(x, wb, b2d)


def kernel(x, weight, bias):
    return _linear(x, weight, bias)


# speedup vs baseline: 1.2806x; 1.0544x over previous
import jax
import jax.numpy as jnp
from jax import lax
from jax.experimental import pallas as pl
from jax.experimental.pallas import tpu as pltpu


def _tile_body(x_ref, w_ref, b_ref, o_ref):
    o_ref[...] = (
        lax.dot_general(
            x_ref[...].astype(jnp.bfloat16),
            w_ref[...].astype(jnp.bfloat16),
            dimension_numbers=(((1,), (1,)), ((), ())),
            preferred_element_type=jnp.float32,
        )
        + b_ref[...]
    )


@jax.jit
def _linear(x, weight, bias):
    M, K = x.shape
    N, Kw = weight.shape
    assert K == Kw, "weight inner dim must match x"

    b2d = bias.reshape(1, N).astype(jnp.float32)

    bm = 512 if M % 512 == 0 else M
    bn = 2048 if N % 2048 == 0 else N
    grid = (N // bn, M // bm)

    cost = pl.CostEstimate(
        flops=2 * M * N * K,
        transcendentals=0,
        bytes_accessed=4 * (M * K * (N // bn) + N * K + M * N + N),
    )

    return pl.pallas_call(
        _tile_body,
        out_shape=jax.ShapeDtypeStruct((M, N), jnp.float32),
        grid=grid,
        in_specs=[
            pl.BlockSpec((bm, K), lambda j, i: (i, 0)),
            pl.BlockSpec(
                (bn, K),
                lambda j, i: (j, 0),
                pipeline_mode=pl.Buffered(buffer_count=1),
            ),
            pl.BlockSpec((1, bn), lambda j, i: (0, j)),
        ],
        out_specs=pl.BlockSpec((bm, bn), lambda j, i: (i, j)),
        compiler_params=pltpu.CompilerParams(
            dimension_semantics=("parallel", "arbitrary"),
            vmem_limit_bytes=61 * 1024 * 1024,
        ),
        cost_estimate=cost,
    )(x, weight, b2d)


def kernel(x, weight, bias):
    return _linear(x, weight, bias)
